# baseline (speedup 1.0000x reference)
import jax
import jax.numpy as jnp
from jax.experimental import pallas as pl
from jax.experimental.pallas import tpu as pltpu

_GROUP = 32

_QDTYPE = jnp.float8_e5m2

_BMQ = 4096
_BKQ = 256

_BM = 1024
_BN = 1024

_MAGIC_HALF = 1.5 * 2.0**23 * 0.5
_MAGIC_ONE = 1.5 * 2.0**23 * 1.0
_MAGIC_TWO = 1.5 * 2.0**23 * 2.0


def _qdq(rot):
    g = rot.shape[0] // _GROUP
    r3 = rot.reshape(g, _GROUP, rot.shape[-1])
    amax = jnp.max(jnp.abs(r3), axis=1, keepdims=True)
    amax = jnp.maximum(amax, 1e-30)
    ebits = jax.lax.shift_right_logical(
        jax.lax.bitcast_convert_type(amax, jnp.int32), 23)
    scale = jax.lax.bitcast_convert_type((ebits - 2) << 23, jnp.float32)
    inv_scale = jax.lax.bitcast_convert_type((256 - ebits) << 23, jnp.float32)
    u = r3 * inv_scale
    au = jnp.abs(u)
    u6 = jnp.clip(u, -6.0, 6.0)
    m = jnp.where(au < 2.0, _MAGIC_HALF,
                  jnp.where(au < 4.0, _MAGIC_ONE, _MAGIC_TWO))
    q = (u6 + m) - m
    return (q * scale).reshape(rot.shape)


def _quant_body(a_ref, bdh_ref, o_ref):
    ab = a_ref[...].astype(jnp.bfloat16)
    rot = jax.lax.dot_general(
        bdh_ref[...], ab, (((1,), (1,)), ((), ())),
        preferred_element_type=jnp.float32)
    o_ref[...] = _qdq(rot).astype(o_ref.dtype)


def _quant_rotate_t(a, bdh):
    r, k = a.shape
    return pl.pallas_call(
        _quant_body,
        grid=(r // _BMQ, k // _BKQ),
        in_specs=[
            pl.BlockSpec((_BMQ, _BKQ), lambda i, j: (i, j)),
            pl.BlockSpec((_BKQ, _BKQ), lambda i, j: (0, 0)),
        ],
        out_specs=pl.BlockSpec((_BKQ, _BMQ), lambda i, j: (j, i)),
        out_shape=jax.ShapeDtypeStruct((k, r), _QDTYPE),
        compiler_params=pltpu.CompilerParams(
            dimension_semantics=("parallel", "arbitrary"),
            vmem_limit_bytes=56 * 1024 * 1024,
        ),
        name="rot_quant_t",
    )(a, bdh)


def kernel(x, weight, bias, hadamard):
    k = x.shape[1]
    tokens = x.shape[0]
    out_f = weight.shape[0]
    bdh = jnp.kron(jnp.eye(_BKQ // _GROUP, dtype=hadamard.dtype),
                   hadamard).astype(jnp.bfloat16)
    xt_dq = _quant_rotate_t(x, bdh)

    nj = out_f // _BN
    ni = tokens // _BM
    wrows = _BN // ni
    nkc = k // _BKQ

    def _fused_body(xt_ref, w_ref, bdh_ref, b_ref, o_ref, wq_ref):
        g = pl.program_id(0)
        i = pl.program_id(1)

        slot2 = jax.lax.rem(g + 1, 2)
        wq = jnp.concatenate(
            [wq_ref[slot2, q] for q in range(ni)], axis=1)
        acc = jax.lax.dot_general(
            xt_ref[...], wq, (((0,), (0,)), ((), ())),
            preferred_element_type=jnp.float32)
        o_ref[...] = acc + b_ref[...]

        slot = jax.lax.rem(g, 2)
        wb = w_ref[...].astype(jnp.bfloat16)
        for kc in range(nkc):
            rot = jax.lax.dot_general(
                bdh_ref[...], wb[:, kc * _BKQ:(kc + 1) * _BKQ],
                (((1,), (1,)), ((), ())),
                preferred_element_type=jnp.float32)
            wq_ref[slot, i, kc * _BKQ:(kc + 1) * _BKQ, :] = (
                _qdq(rot).astype(wq_ref.dtype))

    return pl.pallas_call(
        _fused_body,
        grid=(nj + 1, ni),
        in_specs=[
            pl.BlockSpec((k, _BM), lambda g, i: (0, i)),
            pl.BlockSpec((wrows, k),
                         lambda g, i: (jnp.minimum(g, nj - 1) * ni + i, 0)),
            pl.BlockSpec((_BKQ, _BKQ), lambda g, i: (0, 0)),
            pl.BlockSpec((1, _BN),
                         lambda g, i: (0, jnp.maximum(g - 1, 0))),
        ],
        out_specs=pl.BlockSpec(
            (_BM, _BN), lambda g, i: (i, jnp.maximum(g - 1, 0))),
        out_shape=jax.ShapeDtypeStruct((tokens, out_f), jnp.float32),
        scratch_shapes=[pltpu.VMEM((2, ni, k, wrows), _QDTYPE)],
        compiler_params=pltpu.CompilerParams(
            dimension_semantics=("arbitrary", "arbitrary"),
            vmem_limit_bytes=56 * 1024 * 1024,
        ),
        name="dq_matmul_bias",
    )(xt_dq, weight, bdh, bias.reshape(1, -1))


# speedup vs baseline: 1.0090x; 1.0090x over previous
import jax
import jax.numpy as jnp
from jax.experimental import pallas as pl
from jax.experimental.pallas import tpu as pltpu

_GROUP = 32

_QDTYPE = jnp.float8_e5m2

_BMQ = 4096
_BKQ = 256

_BM = 1024
_BN = 1024

_MAGIC_HALF = 1.5 * 2.0**23 * 0.5
_MAGIC_ONE = 1.5 * 2.0**23 * 1.0
_MAGIC_TWO = 1.5 * 2.0**23 * 2.0


def _qdq(rot):
    g = rot.shape[0] // _GROUP
    r3 = rot.reshape(g, _GROUP, rot.shape[-1])
    amax = jnp.max(jnp.abs(r3), axis=1, keepdims=True)
    amax = jnp.maximum(amax, 1e-30)
    ebits = jax.lax.shift_right_logical(
        jax.lax.bitcast_convert_type(amax, jnp.int32), 23)
    scale = jax.lax.bitcast_convert_type((ebits - 2) << 23, jnp.float32)
    inv_scale = jax.lax.bitcast_convert_type((256 - ebits) << 23, jnp.float32)
    u = r3 * inv_scale
    au = jnp.abs(u)
    u6 = jnp.clip(u, -6.0, 6.0)
    m = jnp.where(au < 2.0, _MAGIC_HALF,
                  jnp.where(au < 4.0, _MAGIC_ONE, _MAGIC_TWO))
    q = (u6 + m) - m
    return (q * scale).reshape(rot.shape)


def _quant_body(a_ref, bdh_ref, o_ref):
    ab = a_ref[...].astype(jnp.bfloat16)
    rot = jax.lax.dot_general(
        bdh_ref[...], ab, (((1,), (1,)), ((), ())),
        preferred_element_type=jnp.float32)
    o_ref[...] = _qdq(rot).astype(o_ref.dtype)


def _quant_rotate_t(a, bdh):
    r, k = a.shape
    return pl.pallas_call(
        _quant_body,
        grid=(r // _BMQ, k // _BKQ),
        in_specs=[
            pl.BlockSpec((_BMQ, _BKQ), lambda i, j: (i, j)),
            pl.BlockSpec((_BKQ, _BKQ), lambda i, j: (0, 0)),
        ],
        out_specs=pl.BlockSpec((_BKQ, _BMQ), lambda i, j: (j, i)),
        out_shape=jax.ShapeDtypeStruct((k, r), _QDTYPE),
        compiler_params=pltpu.CompilerParams(
            dimension_semantics=("parallel", "arbitrary"),
            vmem_limit_bytes=60 * 1024 * 1024,
        ),
        name="rot_quant_t",
    )(a, bdh)


def kernel(x, weight, bias, hadamard):
    k = x.shape[1]
    tokens = x.shape[0]
    out_f = weight.shape[0]
    bdh = jnp.kron(jnp.eye(_BKQ // _GROUP, dtype=hadamard.dtype),
                   hadamard).astype(jnp.bfloat16)
    xt_dq = _quant_rotate_t(x, bdh)

    nj = out_f // _BN
    ni = tokens // _BM
    wrows = _BN // ni
    nkc = k // _BKQ

    def _fused_body(xt_ref, w_ref, bdh_ref, b_ref, o_ref, wq_ref, stg_ref):
        g = pl.program_id(0)
        i = pl.program_id(1)

        par = i & 1
        gp = g - jnp.where(i == 0, 1, 0)
        slot_prev = gp & 1
        ip = (i + ni - 1) & (ni - 1)
        wq_ref[slot_prev, ip] = stg_ref[1 - par]

        wb = w_ref[...].astype(jnp.bfloat16)
        for kc in range(nkc):
            rot = jax.lax.dot_general(
                bdh_ref[...], wb[:, kc * _BKQ:(kc + 1) * _BKQ],
                (((1,), (1,)), ((), ())),
                preferred_element_type=jnp.float32)
            stg_ref[par, kc * _BKQ:(kc + 1) * _BKQ, :] = (
                _qdq(rot).astype(stg_ref.dtype))

        slot2 = jax.lax.rem(g + 1, 2)
        wq = jnp.concatenate(
            [wq_ref[slot2, q] for q in range(ni)], axis=1)
        acc = jax.lax.dot_general(
            xt_ref[...], wq, (((0,), (0,)), ((), ())),
            preferred_element_type=jnp.float32)
        o_ref[...] = acc + b_ref[...]

    return pl.pallas_call(
        _fused_body,
        grid=(nj + 1, ni),
        in_specs=[
            pl.BlockSpec((k, _BM), lambda g, i: (0, i)),
            pl.BlockSpec((wrows, k),
                         lambda g, i: (jnp.minimum(g, nj - 1) * ni + i, 0)),
            pl.BlockSpec((_BKQ, _BKQ), lambda g, i: (0, 0)),
            pl.BlockSpec((1, _BN),
                         lambda g, i: (0, jnp.maximum(g - 1, 0))),
        ],
        out_specs=pl.BlockSpec(
            (_BM, _BN), lambda g, i: (i, jnp.maximum(g - 1, 0))),
        out_shape=jax.ShapeDtypeStruct((tokens, out_f), jnp.float32),
        scratch_shapes=[pltpu.VMEM((2, ni, k, wrows), _QDTYPE),
                        pltpu.VMEM((2, k, wrows), _QDTYPE)],
        compiler_params=pltpu.CompilerParams(
            dimension_semantics=("arbitrary", "arbitrary"),
            vmem_limit_bytes=60 * 1024 * 1024,
        ),
        name="dq_matmul_bias",
    )(xt_dq, weight, bdh, bias.reshape(1, -1))


# speedup vs baseline: 1.0107x; 1.0017x over previous
import jax
import jax.numpy as jnp
from jax.experimental import pallas as pl
from jax.experimental.pallas import tpu as pltpu

_GROUP = 32

_QDTYPE = jnp.float8_e5m2

_BMQ = 4096
_BKQ = 256

_BM = 1024
_BN = 1024

_MAGIC_HALF = 1.5 * 2.0**23 * 0.5
_MAGIC_ONE = 1.5 * 2.0**23 * 1.0
_MAGIC_TWO = 1.5 * 2.0**23 * 2.0


def _qdq(rot):
    g = rot.shape[0] // _GROUP
    r3 = rot.reshape(g, _GROUP, rot.shape[-1])
    amax = jnp.max(jnp.abs(r3), axis=1, keepdims=True)
    amax = jnp.maximum(amax, 1e-30)
    ebits = jax.lax.shift_right_logical(
        jax.lax.bitcast_convert_type(amax, jnp.int32), 23)
    scale = jax.lax.bitcast_convert_type((ebits - 2) << 23, jnp.float32)
    inv_scale = jax.lax.bitcast_convert_type((256 - ebits) << 23, jnp.float32)
    u = r3 * inv_scale
    au = jnp.abs(u)
    u6 = jnp.clip(u, -6.0, 6.0)
    m = jnp.where(au < 2.0, _MAGIC_HALF,
                  jnp.where(au < 4.0, _MAGIC_ONE, _MAGIC_TWO))
    q = (u6 + m) - m
    return (q * scale).reshape(rot.shape)


def _quant_body(a_ref, bdh_ref, o_ref):
    ab = a_ref[...].astype(jnp.bfloat16)
    rot = jax.lax.dot_general(
        bdh_ref[...], ab, (((1,), (1,)), ((), ())),
        preferred_element_type=jnp.float32)
    o_ref[...] = _qdq(rot).astype(o_ref.dtype)


def _quant_rotate_t(a, bdh, rows=None, bmq=_BMQ):
    r, k = a.shape
    if rows is None:
        rows = r
    return pl.pallas_call(
        _quant_body,
        grid=(rows // bmq, k // _BKQ),
        in_specs=[
            pl.BlockSpec((bmq, _BKQ), lambda i, j: (i, j)),
            pl.BlockSpec((_BKQ, _BKQ), lambda i, j: (0, 0)),
        ],
        out_specs=pl.BlockSpec((_BKQ, bmq), lambda i, j: (j, i)),
        out_shape=jax.ShapeDtypeStruct((k, rows), _QDTYPE),
        compiler_params=pltpu.CompilerParams(
            dimension_semantics=("parallel", "arbitrary"),
            vmem_limit_bytes=60 * 1024 * 1024,
        ),
        name="rot_quant_t",
    )(a, bdh)


def kernel(x, weight, bias, hadamard):
    k = x.shape[1]
    tokens = x.shape[0]
    out_f = weight.shape[0]
    bdh = jnp.kron(jnp.eye(_BKQ // _GROUP, dtype=hadamard.dtype),
                   hadamard).astype(jnp.bfloat16)
    xt_dq = _quant_rotate_t(x, bdh)

    nj = out_f // _BN
    ni = tokens // _BM
    wrows = _BN // ni
    nkc = k // _BKQ

    def _fused_body(xt_ref, w_ref, bdh_ref, b_ref, o_ref, wq_ref, stg_ref):
        g = pl.program_id(0)
        i = pl.program_id(1)

        par = i & 1
        gp = g - jnp.where(i == 0, 1, 0)
        slot_prev = gp & 1
        ip = (i + ni - 1) & (ni - 1)
        wq_ref[slot_prev, ip] = stg_ref[1 - par]

        wb = w_ref[...].astype(jnp.bfloat16)
        for kc in range(nkc):
            rot = jax.lax.dot_general(
                bdh_ref[...], wb[:, kc * _BKQ:(kc + 1) * _BKQ],
                (((1,), (1,)), ((), ())),
                preferred_element_type=jnp.float32)
            stg_ref[par, kc * _BKQ:(kc + 1) * _BKQ, :] = (
                _qdq(rot).astype(stg_ref.dtype))

        slot2 = jax.lax.rem(g + 1, 2)
        wq = jnp.concatenate(
            [wq_ref[slot2, q] for q in range(ni)], axis=1)
        acc = jax.lax.dot_general(
            xt_ref[...], wq, (((0,), (0,)), ((), ())),
            preferred_element_type=jnp.float32)
        o_ref[...] = acc + b_ref[...]

    return pl.pallas_call(
        _fused_body,
        grid=(nj + 1, ni),
        in_specs=[
            pl.BlockSpec((k, _BM), lambda g, i: (0, i)),
            pl.BlockSpec((wrows, k),
                         lambda g, i: (jnp.minimum(g, nj - 1) * ni + i, 0)),
            pl.BlockSpec((_BKQ, _BKQ), lambda g, i: (0, 0)),
            pl.BlockSpec((1, _BN),
                         lambda g, i: (0, jnp.maximum(g - 1, 0))),
        ],
        out_specs=pl.BlockSpec(
            (_BM, _BN), lambda g, i: (i, jnp.maximum(g - 1, 0))),
        out_shape=jax.ShapeDtypeStruct((tokens, out_f), jnp.float32),
        scratch_shapes=[pltpu.VMEM((2, ni, k, wrows), _QDTYPE),
                        pltpu.VMEM((2, k, wrows), _QDTYPE)],
        compiler_params=pltpu.CompilerParams(
            dimension_semantics=("arbitrary", "arbitrary"),
            vmem_limit_bytes=60 * 1024 * 1024,
        ),
        name="dq_matmul_bias",
    )(xt_dq, weight, bdh, bias.reshape(1, -1))
